# Initial kernel scaffold; baseline (speedup 1.0000x reference)
import jax
import jax.numpy as jnp
from jax.experimental import pallas as pl


def kernel(x, positive_w, negative_w, V_weight, U_weight):
    raise NotImplementedError("write your pallas kernel here")


# speedup vs baseline: 5.4269x; 5.4269x over previous
import dataclasses
import functools

import jax
import jax.numpy as jnp
from jax import lax
from jax.experimental import pallas as pl
from jax.experimental.pallas import tpu as pltpu
from jax.experimental.pallas import tpu_sc as plsc

B = 16384
D = 64
C_POS = 20
C_NEG = 50
C = C_POS + C_NEG
C_PAD = 80
NW = 32
BPW = B // NW
NB = 8
STEPS = BPW // NB
ROWS_STEP = NB * C
GCHUNK = 112
NGC = ROWS_STEP // GCHUNK


def _sc_body(u_hbm, v_hbm, idx_hbm, x_hbm, out_hbm,
             xbuf, vcs, idx_v, rows, out_v, sem):
    wid = lax.axis_index("s") * 2 + lax.axis_index("c")
    base = wid * BPW

    lane = lax.iota(jnp.int32, 16)
    masks = [lane == j for j in range(16)]

    pltpu.sync_copy(x_hbm.at[pl.ds(base, BPW)], xbuf)
    vc_copies = [
        pltpu.async_copy(
            v_hbm.at[xbuf.at[pl.ds(k * 128, 128)]],
            vcs.at[pl.ds(k * 128, 128)], sem)
        for k in range(BPW // 128)
    ]
    for cp in vc_copies:
        cp.wait()

    @pl.loop(0, STEPS)
    def _step(s):
        b0 = base + s * NB
        pltpu.sync_copy(idx_hbm.at[pl.ds(b0 * C, ROWS_STEP)], idx_v)
        u_copies = [
            pltpu.async_copy(
                u_hbm.at[idx_v.at[pl.ds(k * GCHUNK, GCHUNK)]],
                rows.at[pl.ds(k * GCHUNK, GCHUNK)], sem)
            for k in range(NGC)
        ]
        for cp in u_copies:
            cp.wait()

        @pl.loop(0, NB)
        def _row(i):
            bb = s * NB + i
            vc0 = vcs[bb, pl.ds(0, 16)]
            vc1 = vcs[bb, pl.ds(16, 16)]
            vc2 = vcs[bb, pl.ds(32, 16)]
            vc3 = vcs[bb, pl.ds(48, 16)]
            accs = [jnp.zeros((16,), jnp.float32) for _ in range(5)]
            for j in range(C):
                r = i * C + j
                t = rows[r, pl.ds(0, 16)] * vc0
                t = t + rows[r, pl.ds(16, 16)] * vc1
                t = t + rows[r, pl.ds(32, 16)] * vc2
                t = t + rows[r, pl.ds(48, 16)] * vc3
                sv = jnp.sum(t)
                g, l = divmod(j, 16)
                accs[g] = jnp.where(masks[l], sv, accs[g])
            for g in range(5):
                out_v[i, pl.ds(g * 16, 16)] = accs[g]

        pltpu.sync_copy(out_v, out_hbm.at[pl.ds(b0, NB)])


@jax.jit
def _sc_dots(u_weight, v_weight, idx_all, x):
    mesh = plsc.VectorSubcoreMesh(core_axis_name="c", subcore_axis_name="s")
    cp = pltpu.CompilerParams()
    if "needs_layout_passes" in pltpu.CompilerParams.__dataclass_fields__:
        cp = dataclasses.replace(cp, needs_layout_passes=False)
    if "use_tc_tiling_on_sc" in pltpu.CompilerParams.__dataclass_fields__:
        cp = dataclasses.replace(cp, use_tc_tiling_on_sc=False)
    kern = pl.kernel(
        _sc_body,
        out_type=jax.ShapeDtypeStruct((B, C_PAD), jnp.float32),
        mesh=mesh,
        scratch_types=[
            pltpu.VMEM((BPW,), jnp.int32),
            pltpu.VMEM((BPW, D), jnp.float32),
            pltpu.VMEM((ROWS_STEP,), jnp.int32),
            pltpu.VMEM((ROWS_STEP, D), jnp.float32),
            pltpu.VMEM((NB, C_PAD), jnp.float32),
            pltpu.SemaphoreType.DMA,
        ],
        compiler_params=cp,
    )
    return kern(u_weight, v_weight, idx_all, x)


def _tc_body(uv_ref, o_ref):
    z = uv_ref[...]
    col = lax.broadcasted_iota(jnp.int32, z.shape, 1)
    pos = col < C_POS
    valid = col < C
    zs = jnp.where(pos, z, -z)
    ls = jnp.minimum(zs, 0.0) - jnp.log1p(jnp.exp(-jnp.abs(zs)))
    contrib = jnp.where(valid, ls, 0.0)
    o_ref[...] = -jnp.sum(contrib, axis=1)


@jax.jit
def _tc_epilogue(uv):
    blk = 2048
    return pl.pallas_call(
        _tc_body,
        grid=(B // blk,),
        in_specs=[pl.BlockSpec((blk, C_PAD), lambda i: (i, 0))],
        out_specs=pl.BlockSpec((blk,), lambda i: (i,)),
        out_shape=jax.ShapeDtypeStruct((B,), jnp.float32),
    )(uv)


def kernel(x, positive_w, negative_w, V_weight, U_weight):
    idx_all = jnp.concatenate(
        [positive_w.astype(jnp.int32), negative_w.astype(jnp.int32)], axis=1
    ).reshape(-1)
    uv = _sc_dots(U_weight, V_weight, idx_all, x.astype(jnp.int32))
    return _tc_epilogue(uv)
